# baseline (speedup 1.0000x reference)
import functools

import jax
import jax.numpy as jnp
from jax import lax
from jax.experimental import pallas as pl
from jax.experimental.pallas import tpu as pltpu
from jax.experimental.pallas import tpu_sc as plsc

_M_LO_BITS = 0x3A83126F
_M_HI_BITS = 0x3F7FFFFF
_MASK_RANGE = _M_HI_BITS - (_M_LO_BITS + 1)

_B = 8
_N = 147456
_NW = 32
_CPS = _NW // _B
_CHUNK = _N // _CPS
_NBINS = 8192


def _sc_hist_body(gt_hbm, out_hbm, data_v, h0):
    wid = lax.axis_index("c") * 16 + lax.axis_index("s")
    sample = wid // _CPS
    chunk = wid % _CPS
    pltpu.sync_copy(gt_hbm.at[sample, pl.ds(chunk * _CHUNK, _CHUNK)], data_v)

    zeros16 = jnp.zeros((16,), jnp.int32)

    @plsc.parallel_loop(0, _NBINS // 16, unroll=8)
    def _(i):
        h0[pl.ds(i * 16, 16)] = zeros16

    ones16 = jnp.ones((16,), jnp.int32)
    scale = jnp.float32(_NBINS)

    @plsc.parallel_loop(0, _CHUNK // 16, unroll=8)
    def _(i):
        v = data_v[pl.ds(i * 16, 16)]
        bins = jnp.minimum((v * scale).astype(jnp.int32),
                           jnp.int32(_NBINS - 1))
        plsc.addupdate_scatter(h0, [bins], ones16, mask=v > 0.0)

    pltpu.sync_copy(h0, out_hbm.at[wid])


@functools.cache
def _sc_hist():
    return pl.kernel(
        _sc_hist_body,
        out_type=jax.ShapeDtypeStruct((_NW, _NBINS), jnp.int32),
        mesh=plsc.VectorSubcoreMesh(core_axis_name="c", subcore_axis_name="s"),
        scratch_types=[
            pltpu.VMEM((_CHUNK,), jnp.float32),
            pltpu.VMEM((_NBINS,), jnp.int32),
        ],
        compiler_params=pltpu.CompilerParams(needs_layout_passes=False),
    )


def _loss_body(pred_ref, gt_ref, hist_ref, out_ref):
    g = gt_ref[...]
    h = jnp.sum(hist_ref[...], axis=1)
    npos = jnp.sum(h, axis=1)

    lo_trim = npos // 10
    r_lo = jnp.maximum(lo_trim, 1)
    r_hi = npos - lo_trim + 1

    bin_iota = lax.broadcasted_iota(jnp.int32, h.shape, 1)

    def cum_le(b):
        c = jnp.where(bin_iota <= b[:, None], h, 0)
        return jnp.sum(c, axis=1)

    def step(_, carry):
        lo1, hi1, lo2, hi2 = carry
        m1 = lo1 + (hi1 - lo1) // 2
        m2 = lo2 + (hi2 - lo2) // 2
        c1 = cum_le(m1)
        c2 = cum_le(m2)
        ok1 = c1 >= r_lo
        ok2 = c2 >= r_hi
        return (jnp.where(ok1, lo1, m1 + 1), jnp.where(ok1, m1, hi1),
                jnp.where(ok2, lo2, m2 + 1), jnp.where(ok2, m2, hi2))

    B = g.shape[0]
    z = jnp.zeros((B,), jnp.int32)
    f = jnp.full((B,), _NBINS - 1, jnp.int32)
    b1, _, b2, _ = lax.fori_loop(0, 13, step, (z, f, z, f))
    binw = jnp.float32(1.0 / _NBINS)
    t1 = (b1 + 1).astype(jnp.float32) * binw
    t2 = (b2 + 1).astype(jnp.float32) * binw

    gg = g * g
    t1e = t1[:, None, None]
    t2e = t2[:, None, None]
    zero = jnp.float32(0.0)
    rb = jnp.sum(jnp.maximum(t1e - g, zero), axis=(1, 2))
    rbq = jnp.sum(jnp.maximum(t1e * t1e - gg, zero), axis=(1, 2))
    rt = jnp.sum(jnp.maximum(g - t2e, zero), axis=(1, 2))
    rtq = jnp.sum(jnp.maximum(gg - t2e * t2e, zero), axis=(1, 2))
    s_all = jnp.sum(g, axis=(1, 2))
    q_all = jnp.sum(gg, axis=(1, 2))

    lof = lo_trim.astype(jnp.float32)
    nzero = (jnp.float32(g.shape[1] * g.shape[2]) - npos.astype(jnp.float32))
    lz = lof + nzero
    has_trim = lo_trim > 0
    sum_bot = jnp.where(has_trim, lz * t1 - rb, 0.0)
    sq_bot = jnp.where(has_trim, lz * t1 * t1 - rbq, 0.0)
    sum_top = jnp.where(has_trim, lof * t2 + rt, 0.0)
    sq_top = jnp.where(has_trim, lof * t2 * t2 + rtq, 0.0)

    m = npos - 2 * lo_trim
    mf = m.astype(jnp.float32)
    kept_sum = s_all - sum_bot - sum_top
    kept_sq = q_all - sq_bot - sq_top
    mean_t = kept_sum / mf
    var_t = (kept_sq - mf * mean_t * mean_t) / jnp.maximum(mf - 1.0, 1.0)
    std_t = jnp.sqrt(jnp.maximum(var_t, 0.0))
    has_enough = npos >= 10
    mean = jnp.where(has_enough, mean_t, 0.0)
    std = jnp.where(has_enough, std_t, 1.0)

    p = pred_ref[...]
    gu1 = lax.bitcast_convert_type(g, jnp.uint32) - jnp.uint32(1)
    inv = (1.0 / (std + 1e-8))[:, None, None]
    gtr = (g - mean[:, None, None]) * inv
    d = jnp.abs(gtr - p)
    d2 = jnp.abs(jnp.tanh(0.1 * gtr) - jnp.tanh(0.1 * p))
    msk = (gu1 - jnp.uint32(_M_LO_BITS)) <= jnp.uint32(_MASK_RANGE)
    tot = jnp.sum(jnp.where(msk, d + d2, 0.0), axis=(1, 2))
    msum = jnp.sum(msk.astype(jnp.int32), axis=(1, 2))
    loss = tot / msum.astype(jnp.float32)
    out_ref[...] = jnp.broadcast_to(loss[:, None], out_ref.shape)


def kernel(pred, gt):
    B = gt.shape[0]
    H, W = gt.shape[-2], gt.shape[-1]
    g3 = gt.reshape(B, H, W)
    p3 = pred.reshape(B, H, W)
    hist = _sc_hist()(gt.reshape(B, H * W))
    h3 = hist.reshape(B, _CPS, _NBINS)
    out = pl.pallas_call(
        _loss_body,
        out_shape=jax.ShapeDtypeStruct((B, 128), jnp.float32),
    )(p3, g3, h3)
    return out[:, 0]


# speedup vs baseline: 1.0417x; 1.0417x over previous
import functools

import jax
import jax.numpy as jnp
from jax import lax
from jax.experimental import pallas as pl
from jax.experimental.pallas import tpu as pltpu
from jax.experimental.pallas import tpu_sc as plsc

_M_LO_BITS = 0x3A83126F
_M_HI_BITS = 0x3F7FFFFF
_MASK_RANGE = _M_HI_BITS - (_M_LO_BITS + 1)

_B = 8
_H = 384
_W = 384
_N = _H * _W
_NW = 32
_CPS = _NW // _B
_ROWS = _H // _CPS
_NBINS = 8192


def _sc_hist_body(gt_hbm, out_hbm, data_v, h0):
    wid = lax.axis_index("c") * 16 + lax.axis_index("s")
    sample = wid // _CPS
    chunk = wid % _CPS
    pltpu.sync_copy(gt_hbm.at[sample, pl.ds(chunk * _ROWS, _ROWS)], data_v)

    zeros16 = jnp.zeros((16,), jnp.int32)

    @plsc.parallel_loop(0, _NBINS // 16, unroll=8)
    def _(i):
        h0[pl.ds(i * 16, 16)] = zeros16

    ones16 = jnp.ones((16,), jnp.int32)
    scale = jnp.float32(_NBINS)

    @plsc.parallel_loop(0, _ROWS, unroll=2)
    def _(i):
        for j in range(_W // 16):
            v = data_v[i, pl.ds(j * 16, 16)]
            bins = jnp.minimum((v * scale).astype(jnp.int32),
                               jnp.int32(_NBINS - 1))
            plsc.addupdate_scatter(h0, [bins], ones16, mask=v > 0.0)

    pltpu.sync_copy(h0, out_hbm.at[wid])


@functools.cache
def _sc_hist():
    return pl.kernel(
        _sc_hist_body,
        out_type=jax.ShapeDtypeStruct((_NW, _NBINS), jnp.int32),
        mesh=plsc.VectorSubcoreMesh(core_axis_name="c", subcore_axis_name="s"),
        scratch_types=[
            pltpu.VMEM((_ROWS, _W), jnp.float32),
            pltpu.VMEM((_NBINS,), jnp.int32),
        ],
        compiler_params=pltpu.CompilerParams(needs_layout_passes=False),
    )


def _loss_body(pred_ref, gt_ref, hist_ref, out_ref):
    g = gt_ref[...]
    h = jnp.sum(hist_ref[...], axis=1)
    npos = jnp.sum(h, axis=1)

    lo_trim = npos // 10
    r_lo = jnp.maximum(lo_trim, 1)
    r_hi = npos - lo_trim + 1

    bin_iota = lax.broadcasted_iota(jnp.int32, h.shape, 1)

    def cum_le(b):
        c = jnp.where(bin_iota <= b[:, None], h, 0)
        return jnp.sum(c, axis=1)

    def step(_, carry):
        lo1, hi1, lo2, hi2 = carry
        m1 = lo1 + (hi1 - lo1) // 2
        m2 = lo2 + (hi2 - lo2) // 2
        c1 = cum_le(m1)
        c2 = cum_le(m2)
        ok1 = c1 >= r_lo
        ok2 = c2 >= r_hi
        return (jnp.where(ok1, lo1, m1 + 1), jnp.where(ok1, m1, hi1),
                jnp.where(ok2, lo2, m2 + 1), jnp.where(ok2, m2, hi2))

    B = g.shape[0]
    z = jnp.zeros((B,), jnp.int32)
    f = jnp.full((B,), _NBINS - 1, jnp.int32)
    b1, _, b2, _ = lax.fori_loop(0, 13, step, (z, f, z, f))
    binw = jnp.float32(1.0 / _NBINS)
    t1 = (b1 + 1).astype(jnp.float32) * binw
    t2 = (b2 + 1).astype(jnp.float32) * binw

    gg = g * g
    t1e = t1[:, None, None]
    t2e = t2[:, None, None]
    zero = jnp.float32(0.0)
    rb = jnp.sum(jnp.maximum(t1e - g, zero), axis=(1, 2))
    rbq = jnp.sum(jnp.maximum(t1e * t1e - gg, zero), axis=(1, 2))
    rt = jnp.sum(jnp.maximum(g - t2e, zero), axis=(1, 2))
    rtq = jnp.sum(jnp.maximum(gg - t2e * t2e, zero), axis=(1, 2))
    s_all = jnp.sum(g, axis=(1, 2))
    q_all = jnp.sum(gg, axis=(1, 2))

    lof = lo_trim.astype(jnp.float32)
    nzero = (jnp.float32(g.shape[1] * g.shape[2]) - npos.astype(jnp.float32))
    lz = lof + nzero
    has_trim = lo_trim > 0
    sum_bot = jnp.where(has_trim, lz * t1 - rb, 0.0)
    sq_bot = jnp.where(has_trim, lz * t1 * t1 - rbq, 0.0)
    sum_top = jnp.where(has_trim, lof * t2 + rt, 0.0)
    sq_top = jnp.where(has_trim, lof * t2 * t2 + rtq, 0.0)

    m = npos - 2 * lo_trim
    mf = m.astype(jnp.float32)
    kept_sum = s_all - sum_bot - sum_top
    kept_sq = q_all - sq_bot - sq_top
    mean_t = kept_sum / mf
    var_t = (kept_sq - mf * mean_t * mean_t) / jnp.maximum(mf - 1.0, 1.0)
    std_t = jnp.sqrt(jnp.maximum(var_t, 0.0))
    has_enough = npos >= 10
    mean = jnp.where(has_enough, mean_t, 0.0)
    std = jnp.where(has_enough, std_t, 1.0)

    p = pred_ref[...]
    gu1 = lax.bitcast_convert_type(g, jnp.uint32) - jnp.uint32(1)
    inv = (1.0 / (std + 1e-8))[:, None, None]
    gtr = (g - mean[:, None, None]) * inv
    d = jnp.abs(gtr - p)
    d2 = jnp.abs(jnp.tanh(0.1 * gtr) - jnp.tanh(0.1 * p))
    msk = (gu1 - jnp.uint32(_M_LO_BITS)) <= jnp.uint32(_MASK_RANGE)
    tot = jnp.sum(jnp.where(msk, d + d2, 0.0), axis=(1, 2))
    msum = jnp.sum(msk.astype(jnp.int32), axis=(1, 2))
    loss = tot / msum.astype(jnp.float32)
    out_ref[...] = jnp.broadcast_to(loss[:, None], out_ref.shape)


def kernel(pred, gt):
    B = gt.shape[0]
    H, W = gt.shape[-2], gt.shape[-1]
    g3 = gt.reshape(B, H, W)
    p3 = pred.reshape(B, H, W)
    hist = _sc_hist()(g3)
    h3 = hist.reshape(B, _CPS, _NBINS)
    out = pl.pallas_call(
        _loss_body,
        out_shape=jax.ShapeDtypeStruct((B, 128), jnp.float32),
    )(p3, g3, h3)
    return out[:, 0]
